# baseline (speedup 1.0000x reference)
import jax
import jax.numpy as jnp
from jax.experimental import pallas as pl
from jax.experimental.pallas import tpu as pltpu

HIDDEN_SIZE = 4096
N_EXPERTS = 64
BLOCK_M = 512
NBUF = 3


def _copy(x_hbm, xbuf, sems, chunk, slot):
    return pltpu.make_async_copy(
        x_hbm.at[pl.ds(chunk * BLOCK_M, BLOCK_M), :],
        xbuf.at[slot],
        sems.at[slot],
    )


def _router_kernel(x_hbm, w_ref, o_ref, xbuf, sems):
    nchunks = x_hbm.shape[0] // BLOCK_M
    for s in range(min(NBUF, nchunks)):
        _copy(x_hbm, xbuf, sems, s, s).start()
    for i in range(nchunks):
        slot = i % NBUF
        _copy(x_hbm, xbuf, sems, i, slot).wait()
        o_ref[pl.ds(i * BLOCK_M, BLOCK_M), :] = jax.lax.dot_general(
            xbuf[slot], w_ref[...],
            dimension_numbers=(((1,), (1,)), ((), ())),
            preferred_element_type=jnp.float32,
        )
        nxt = i + NBUF
        if nxt < nchunks:
            _copy(x_hbm, xbuf, sems, nxt, slot).start()


def kernel(hidden_states, weight):
    hs = hidden_states.reshape(-1, HIDDEN_SIZE)
    m = hs.shape[0]
    return pl.pallas_call(
        _router_kernel,
        in_specs=[
            pl.BlockSpec(memory_space=pl.ANY),
            pl.BlockSpec(memory_space=pltpu.VMEM),
        ],
        out_specs=pl.BlockSpec(memory_space=pltpu.VMEM),
        out_shape=jax.ShapeDtypeStruct((m, N_EXPERTS), jnp.float32),
        scratch_shapes=[
            pltpu.VMEM((NBUF, BLOCK_M, HIDDEN_SIZE), jnp.float32),
            pltpu.SemaphoreType.DMA((NBUF,)),
        ],
    )(hs, weight)


# speedup vs baseline: 1.0023x; 1.0023x over previous
import jax
import jax.numpy as jnp
from jax.experimental import pallas as pl
from jax.experimental.pallas import tpu as pltpu

HIDDEN_SIZE = 4096
N_EXPERTS = 64
BLOCK_M = 512
NBUF = 3


HALF = BLOCK_M // 2


def _copies(x_hbm, xbuf, sems, chunk, slot):
    return [
        pltpu.make_async_copy(
            x_hbm.at[pl.ds(chunk * BLOCK_M + h * HALF, HALF), :],
            xbuf.at[slot, pl.ds(h * HALF, HALF)],
            sems.at[slot, h],
        )
        for h in range(2)
    ]


def _router_kernel(x_hbm, w_ref, o_ref, xbuf, sems):
    nchunks = x_hbm.shape[0] // BLOCK_M
    for s in range(min(NBUF, nchunks)):
        for c in _copies(x_hbm, xbuf, sems, s, s):
            c.start()
    for i in range(nchunks):
        slot = i % NBUF
        for c in _copies(x_hbm, xbuf, sems, i, slot):
            c.wait()
        o_ref[pl.ds(i * BLOCK_M, BLOCK_M), :] = jax.lax.dot_general(
            xbuf[slot], w_ref[...],
            dimension_numbers=(((1,), (1,)), ((), ())),
            preferred_element_type=jnp.float32,
        )
        nxt = i + NBUF
        if nxt < nchunks:
            for c in _copies(x_hbm, xbuf, sems, nxt, slot):
                c.start()


def kernel(hidden_states, weight):
    hs = hidden_states.reshape(-1, HIDDEN_SIZE)
    m = hs.shape[0]
    return pl.pallas_call(
        _router_kernel,
        in_specs=[
            pl.BlockSpec(memory_space=pl.ANY),
            pl.BlockSpec(memory_space=pltpu.VMEM),
        ],
        out_specs=pl.BlockSpec(memory_space=pltpu.VMEM),
        out_shape=jax.ShapeDtypeStruct((m, N_EXPERTS), jnp.float32),
        scratch_shapes=[
            pltpu.VMEM((NBUF, BLOCK_M, HIDDEN_SIZE), jnp.float32),
            pltpu.SemaphoreType.DMA((NBUF, 2)),
        ],
    )(hs, weight)


# speedup vs baseline: 1.0250x; 1.0226x over previous
import jax
import jax.numpy as jnp
from jax.experimental import pallas as pl

HIDDEN_SIZE = 4096
N_EXPERTS = 64
BLOCK_M = 512


def _router_kernel(x1_ref, x2_ref, w_ref, o_ref):
    w = w_ref[...]
    o_ref[0] = jax.lax.dot_general(
        x1_ref[0], w,
        dimension_numbers=(((1,), (1,)), ((), ())),
        preferred_element_type=jnp.float32,
    )
    o_ref[1] = jax.lax.dot_general(
        x2_ref[0], w,
        dimension_numbers=(((1,), (1,)), ((), ())),
        preferred_element_type=jnp.float32,
    )


def kernel(hidden_states, weight):
    hs = hidden_states.reshape(2, -1, HIDDEN_SIZE)
    half = hs.shape[1]
    grid = (half // BLOCK_M,)
    out = pl.pallas_call(
        _router_kernel,
        grid=grid,
        in_specs=[
            pl.BlockSpec((1, BLOCK_M, HIDDEN_SIZE), lambda i: (0, i, 0)),
            pl.BlockSpec((1, BLOCK_M, HIDDEN_SIZE), lambda i: (1, i, 0)),
            pl.BlockSpec((N_EXPERTS, HIDDEN_SIZE), lambda i: (0, 0)),
        ],
        out_specs=pl.BlockSpec((2, BLOCK_M, N_EXPERTS), lambda i: (0, i, 0)),
        out_shape=jax.ShapeDtypeStruct((2, half, N_EXPERTS), jnp.float32),
    )(hs, hs, weight)
    return out.reshape(2 * half, N_EXPERTS)


# speedup vs baseline: 1.0659x; 1.0399x over previous
import jax
import jax.numpy as jnp
from jax.experimental import pallas as pl

HIDDEN_SIZE = 4096
N_EXPERTS = 64
BLOCK_M = 512


def _router_kernel(x_ref, w_ref, o_ref):
    o_ref[...] = jax.lax.dot_general(
        x_ref[...].astype(jnp.bfloat16), w_ref[...].astype(jnp.bfloat16),
        dimension_numbers=(((1,), (1,)), ((), ())),
        preferred_element_type=jnp.float32,
    )


def kernel(hidden_states, weight):
    hs = hidden_states.reshape(-1, HIDDEN_SIZE)
    m = hs.shape[0]
    grid = (m // BLOCK_M,)
    return pl.pallas_call(
        _router_kernel,
        grid=grid,
        in_specs=[
            pl.BlockSpec((BLOCK_M, HIDDEN_SIZE), lambda i: (i, 0)),
            pl.BlockSpec((N_EXPERTS, HIDDEN_SIZE), lambda i: (0, 0)),
        ],
        out_specs=pl.BlockSpec((BLOCK_M, N_EXPERTS), lambda i: (i, 0)),
        out_shape=jax.ShapeDtypeStruct((m, N_EXPERTS), jnp.float32),
    )(hs, weight)
